# baseline (speedup 1.0000x reference)
import numpy as np

import jax
import jax.numpy as jnp
from jax import lax
from jax.experimental import pallas as pl
from jax.experimental.pallas import tpu as pltpu

B, T, N = 16, 2048, 512
TC = 512
S = T // TC


def _s32(v):
    return int(np.int32(np.uint32(v)))


_K1 = (_s32(0x6D3E048F), _s32(0x1022172D))
_K2 = (_s32(0x03D7B32D), _s32(0xADD083F4))
_K3 = (_s32(0x92FB20EA), _s32(0x0F38D913))
_K4 = (_s32(0xBAD56946), _s32(0x354BA891))
_PARITY = _s32(0x1BD11BDA)

_KP_MASK = 2516583
_KP_ZERO = 6710887
_KP_RND = 838861

_ROT_A = (13, 15, 26, 6)
_ROT_B = (17, 29, 16, 24)


def _rotl(x, r):
    return lax.shift_left(x, jnp.int32(r)) | lax.shift_right_logical(
        x, jnp.int32(32 - r))


def _threefry(kpair, ctr):
    ks0 = jnp.int32(kpair[0])
    ks1 = jnp.int32(kpair[1])
    ks2 = ks0 ^ ks1 ^ jnp.int32(_PARITY)
    ks = (ks0, ks1, ks2)
    x0 = jnp.full_like(ctr, ks0)
    x1 = ctr + ks1
    rots = (_ROT_A, _ROT_B, _ROT_A, _ROT_B, _ROT_A)
    for i in range(5):
        for r in rots[i]:
            x0 = x0 + x1
            x1 = _rotl(x1, r)
            x1 = x1 ^ x0
        x0 = x0 + ks[(i + 1) % 3]
        x1 = x1 + ks[(i + 2) % 3] + jnp.int32(i + 1)
    return x0 ^ x1


def _bern(bits, k):
    return lax.shift_right_logical(bits, jnp.int32(9)) < jnp.int32(k)


def _u01(bits):
    f = lax.bitcast_convert_type(
        lax.shift_right_logical(bits, jnp.int32(9)) | jnp.int32(0x3F800000),
        jnp.float32)
    return f - jnp.float32(1.0)


def _pass1(x_ref, vals_ref, maskout_ref, rmask_ref, pmax_ref):
    b = pl.program_id(0)
    s = pl.program_id(1)
    x = x_ref[...]
    tt = lax.broadcasted_iota(jnp.int32, (1, TC, N), 1)
    nn = lax.broadcasted_iota(jnp.int32, (1, TC, N), 2)
    c = (b * (T * N) + s * (TC * N)) + tt * jnp.int32(N) + nn
    cn = b * jnp.int32(N) + lax.broadcasted_iota(jnp.int32, (1, 1, N), 2)
    mask_row = _bern(_threefry(_K1, cn), _KP_MASK)
    zero = _bern(_threefry(_K2, c), _KP_ZERO) & mask_row
    feats = jnp.where(zero, jnp.float32(0.0), x)
    rnd = _bern(_threefry(_K3, c), _KP_RND) & mask_row & (~zero)
    u = _u01(_threefry(_K4, c))
    vals_ref[...] = jnp.where(rnd, u, feats)
    maskout_ref[...] = jnp.broadcast_to(mask_row, (1, TC, N)).astype(jnp.int32)
    rmask_ref[...] = rnd.astype(jnp.int8)
    pmax_ref[...] = jnp.full((1, 1, 1), jnp.max(feats), jnp.float32)


def _pass2(pmax_ref, vals_ref, rmask_ref, out_ref):
    m = jnp.max(pmax_ref[...])
    v = vals_ref[...]
    spike = jnp.round(m * v)
    out_ref[...] = jnp.where(rmask_ref[...] != 0, spike, v)


def kernel(features):
    blk = lambda b, s: (b, s, 0)
    vals, mask_out, rmask, pmax = pl.pallas_call(
        _pass1,
        grid=(B, S),
        in_specs=[pl.BlockSpec((1, TC, N), blk)],
        out_specs=[
            pl.BlockSpec((1, TC, N), blk),
            pl.BlockSpec((1, TC, N), blk),
            pl.BlockSpec((1, TC, N), blk),
            pl.BlockSpec((1, 1, 1), lambda b, s: (b * S + s, 0, 0)),
        ],
        out_shape=[
            jax.ShapeDtypeStruct((B, T, N), jnp.float32),
            jax.ShapeDtypeStruct((B, T, N), jnp.int32),
            jax.ShapeDtypeStruct((B, T, N), jnp.int8),
            jax.ShapeDtypeStruct((B * S, 1, 1), jnp.float32),
        ],
        compiler_params=pltpu.CompilerParams(
            dimension_semantics=("parallel", "parallel")),
    )(features)
    out = pl.pallas_call(
        _pass2,
        grid=(B, S),
        in_specs=[
            pl.BlockSpec((B * S, 1, 1), lambda b, s: (0, 0, 0)),
            pl.BlockSpec((1, TC, N), blk),
            pl.BlockSpec((1, TC, N), blk),
        ],
        out_specs=pl.BlockSpec((1, TC, N), blk),
        out_shape=jax.ShapeDtypeStruct((B, T, N), jnp.float32),
        input_output_aliases={1: 0},
        compiler_params=pltpu.CompilerParams(
            dimension_semantics=("parallel", "parallel")),
    )(pmax, vals, rmask)
    return out, mask_out


# speedup vs baseline: 1.6499x; 1.6499x over previous
import numpy as np

import jax
import jax.numpy as jnp
from jax import lax
from jax.experimental import pallas as pl
from jax.experimental.pallas import tpu as pltpu

B, T, N = 16, 2048, 512
TC = 512
S = T // TC


def _s32(v):
    return int(np.int32(np.uint32(v)))


_K1 = (_s32(0x6D3E048F), _s32(0x1022172D))
_K2 = (_s32(0x03D7B32D), _s32(0xADD083F4))
_K3 = (_s32(0x92FB20EA), _s32(0x0F38D913))
_K4 = (_s32(0xBAD56946), _s32(0x354BA891))
_PARITY = _s32(0x1BD11BDA)

_KP_MASK = 2516583
_KP_ZERO = 6710887
_KP_RND = 838861

_ROT_A = (13, 15, 26, 6)
_ROT_B = (17, 29, 16, 24)


def _rotl(x, r):
    return lax.shift_left(x, jnp.int32(r)) | lax.shift_right_logical(
        x, jnp.int32(32 - r))


def _threefry(kpair, ctr):
    ks0 = jnp.int32(kpair[0])
    ks1 = jnp.int32(kpair[1])
    ks2 = ks0 ^ ks1 ^ jnp.int32(_PARITY)
    ks = (ks0, ks1, ks2)
    x0 = jnp.full_like(ctr, ks0)
    x1 = ctr + ks1
    rots = (_ROT_A, _ROT_B, _ROT_A, _ROT_B, _ROT_A)
    for i in range(5):
        for r in rots[i]:
            x0 = x0 + x1
            x1 = _rotl(x1, r)
            x1 = x1 ^ x0
        x0 = x0 + ks[(i + 1) % 3]
        x1 = x1 + ks[(i + 2) % 3] + jnp.int32(i + 1)
    return x0 ^ x1


def _bern(bits, k):
    return lax.shift_right_logical(bits, jnp.int32(9)) < jnp.int32(k)


def _u01(bits):
    f = lax.bitcast_convert_type(
        lax.shift_right_logical(bits, jnp.int32(9)) | jnp.int32(0x3F800000),
        jnp.float32)
    return f - jnp.float32(1.0)


SUB = 8
GRP = 32


def _pass1(x_ref, vals_ref, maskout_ref, rmask_ref, pmax_ref):
    b = pl.program_id(0)
    s = pl.program_id(1)
    cn = b * jnp.int32(N) + lax.broadcasted_iota(jnp.int32, (1, N), 1)
    mask_row = _bern(_threefry(_K1, cn), _KP_MASK)
    mask_sub = jnp.broadcast_to(mask_row, (SUB, N))
    mask_i32 = mask_sub.astype(jnp.int32)
    coff = (lax.broadcasted_iota(jnp.int32, (SUB, N), 0) * jnp.int32(N)
            + lax.broadcasted_iota(jnp.int32, (SUB, N), 1))
    cblock = b * (T * N) + s * (TC * N)

    def group(j, mx):
        rnd_pieces = []
        for k in range(GRP // SUB):
            row0 = j * GRP + k * SUB
            sl = pl.ds(row0, SUB)
            x = x_ref[0, sl, :]
            c = (cblock + row0 * jnp.int32(N)) + coff
            zero = _bern(_threefry(_K2, c), _KP_ZERO) & mask_sub
            feats = jnp.where(zero, jnp.float32(0.0), x)
            rnd = _bern(_threefry(_K3, c), _KP_RND) & mask_sub & (~zero)
            u = _u01(_threefry(_K4, c))
            vals_ref[0, sl, :] = jnp.where(rnd, u, feats)
            maskout_ref[0, sl, :] = mask_i32
            rnd_pieces.append(rnd)
            mx = jnp.maximum(mx, feats)
        rnd_grp = jnp.concatenate(rnd_pieces, axis=0)
        rmask_ref[0, pl.ds(j * GRP, GRP), :] = rnd_grp.astype(jnp.int8)
        return mx

    mx = lax.fori_loop(0, TC // GRP, group,
                       jnp.full((SUB, N), -jnp.inf, jnp.float32))
    pmax_ref[...] = jnp.full((1, 1, 1), jnp.max(mx), jnp.float32)


def _pass2(pmax_ref, vals_ref, rmask_ref, out_ref):
    m = jnp.max(pmax_ref[...])
    v = vals_ref[...]
    spike = jnp.round(m * v)
    out_ref[...] = jnp.where(rmask_ref[...] != 0, spike, v)


def kernel(features):
    blk = lambda b, s: (b, s, 0)
    vals, mask_out, rmask, pmax = pl.pallas_call(
        _pass1,
        grid=(B, S),
        in_specs=[pl.BlockSpec((1, TC, N), blk)],
        out_specs=[
            pl.BlockSpec((1, TC, N), blk),
            pl.BlockSpec((1, TC, N), blk),
            pl.BlockSpec((1, TC, N), blk),
            pl.BlockSpec((1, 1, 1), lambda b, s: (b * S + s, 0, 0)),
        ],
        out_shape=[
            jax.ShapeDtypeStruct((B, T, N), jnp.float32),
            jax.ShapeDtypeStruct((B, T, N), jnp.int32),
            jax.ShapeDtypeStruct((B, T, N), jnp.int8),
            jax.ShapeDtypeStruct((B * S, 1, 1), jnp.float32),
        ],
        compiler_params=pltpu.CompilerParams(
            dimension_semantics=("parallel", "parallel")),
    )(features)
    out = pl.pallas_call(
        _pass2,
        grid=(B, S),
        in_specs=[
            pl.BlockSpec((B * S, 1, 1), lambda b, s: (0, 0, 0)),
            pl.BlockSpec((1, TC, N), blk),
            pl.BlockSpec((1, TC, N), blk),
        ],
        out_specs=pl.BlockSpec((1, TC, N), blk),
        out_shape=jax.ShapeDtypeStruct((B, T, N), jnp.float32),
        input_output_aliases={1: 0},
        compiler_params=pltpu.CompilerParams(
            dimension_semantics=("parallel", "parallel")),
    )(pmax, vals, rmask)
    return out, mask_out


# speedup vs baseline: 1.7099x; 1.0364x over previous
import numpy as np

import jax
import jax.numpy as jnp
from jax import lax
from jax.experimental import pallas as pl
from jax.experimental.pallas import tpu as pltpu

B, T, N = 16, 2048, 512
TC = 512
S = T // TC


def _s32(v):
    return int(np.int32(np.uint32(v)))


_K1 = (_s32(0x6D3E048F), _s32(0x1022172D))
_K2 = (_s32(0x03D7B32D), _s32(0xADD083F4))
_K3 = (_s32(0x92FB20EA), _s32(0x0F38D913))
_K4 = (_s32(0xBAD56946), _s32(0x354BA891))
_PARITY = _s32(0x1BD11BDA)

_KP_MASK = 2516583
_KP_ZERO = 6710887
_KP_RND = 838861

_ROT_A = (13, 15, 26, 6)
_ROT_B = (17, 29, 16, 24)


def _rotl(x, r):
    return lax.shift_left(x, jnp.int32(r)) | lax.shift_right_logical(
        x, jnp.int32(32 - r))


def _wrap32(v):
    return int(np.int32(np.uint32(v & 0xFFFFFFFF)))


def _threefry(kpair, ctr):
    k0 = int(np.uint32(np.int32(kpair[0])))
    k1 = int(np.uint32(np.int32(kpair[1])))
    ks = (k0, k1, k0 ^ k1 ^ 0x1BD11BDA)
    x0 = jnp.full_like(ctr, jnp.int32(_wrap32(ks[0])))
    x1 = ctr + jnp.int32(_wrap32(ks[1]))
    rots = (_ROT_A, _ROT_B, _ROT_A, _ROT_B, _ROT_A)
    for i in range(5):
        for r in rots[i]:
            x0 = x0 + x1
            x1 = _rotl(x1, r)
            x1 = x1 ^ x0
        x0 = x0 + jnp.int32(_wrap32(ks[(i + 1) % 3]))
        x1 = x1 + jnp.int32(_wrap32(ks[(i + 2) % 3] + i + 1))
    return x0 ^ x1


def _bern(bits, k):
    return lax.shift_right_logical(bits, jnp.int32(9)) < jnp.int32(k)


def _u01(bits):
    f = lax.bitcast_convert_type(
        lax.shift_right_logical(bits, jnp.int32(9)) | jnp.int32(0x3F800000),
        jnp.float32)
    return f - jnp.float32(1.0)


SUB = 8
GRP = 32


def _pass1(x_ref, vals_ref, maskout_ref, rmask_ref, pmax_ref):
    b = pl.program_id(0)
    s = pl.program_id(1)
    cn = b * jnp.int32(N) + lax.broadcasted_iota(jnp.int32, (1, N), 1)
    mask_row = _bern(_threefry(_K1, cn), _KP_MASK)
    mask_sub = jnp.broadcast_to(mask_row, (SUB, N))
    mask_i32 = mask_sub.astype(jnp.int32)
    coff = (lax.broadcasted_iota(jnp.int32, (SUB, N), 0) * jnp.int32(N)
            + lax.broadcasted_iota(jnp.int32, (SUB, N), 1))
    cblock = b * (T * N) + s * (TC * N)

    def group(j, mx):
        rnd_pieces = []
        for k in range(GRP // SUB):
            row0 = j * GRP + k * SUB
            sl = pl.ds(row0, SUB)
            x = x_ref[0, sl, :]
            c = (cblock + row0 * jnp.int32(N)) + coff
            zero = _bern(_threefry(_K2, c), _KP_ZERO) & mask_sub
            feats = jnp.where(zero, jnp.float32(0.0), x)
            rnd = _bern(_threefry(_K3, c), _KP_RND) & mask_sub & (~zero)
            u = _u01(_threefry(_K4, c))
            vals_ref[0, sl, :] = jnp.where(rnd, u, feats)
            maskout_ref[0, sl, :] = mask_i32
            rnd_pieces.append(rnd)
            mx = jnp.maximum(mx, feats)
        rnd_grp = jnp.concatenate(rnd_pieces, axis=0)
        rmask_ref[0, pl.ds(j * GRP, GRP), :] = rnd_grp.astype(jnp.int8)
        return mx

    mx = lax.fori_loop(0, TC // GRP, group,
                       jnp.full((SUB, N), -jnp.inf, jnp.float32))
    pmax_ref[...] = jnp.full((1, 1, 1), jnp.max(mx), jnp.float32)


def _pass2(pmax_ref, vals_ref, rmask_ref, out_ref):
    m = jnp.max(pmax_ref[...])
    v = vals_ref[...]
    spike = jnp.round(m * v)
    out_ref[...] = jnp.where(rmask_ref[...] != 0, spike, v)


def kernel(features):
    blk = lambda b, s: (b, s, 0)
    vals, mask_out, rmask, pmax = pl.pallas_call(
        _pass1,
        grid=(B, S),
        in_specs=[pl.BlockSpec((1, TC, N), blk)],
        out_specs=[
            pl.BlockSpec((1, TC, N), blk),
            pl.BlockSpec((1, TC, N), blk),
            pl.BlockSpec((1, TC, N), blk),
            pl.BlockSpec((1, 1, 1), lambda b, s: (b * S + s, 0, 0)),
        ],
        out_shape=[
            jax.ShapeDtypeStruct((B, T, N), jnp.float32),
            jax.ShapeDtypeStruct((B, T, N), jnp.int32),
            jax.ShapeDtypeStruct((B, T, N), jnp.int8),
            jax.ShapeDtypeStruct((B * S, 1, 1), jnp.float32),
        ],
        compiler_params=pltpu.CompilerParams(
            dimension_semantics=("parallel", "parallel")),
    )(features)
    out = pl.pallas_call(
        _pass2,
        grid=(B, S),
        in_specs=[
            pl.BlockSpec((B * S, 1, 1), lambda b, s: (0, 0, 0)),
            pl.BlockSpec((1, TC, N), blk),
            pl.BlockSpec((1, TC, N), blk),
        ],
        out_specs=pl.BlockSpec((1, TC, N), blk),
        out_shape=jax.ShapeDtypeStruct((B, T, N), jnp.float32),
        input_output_aliases={1: 0},
        compiler_params=pltpu.CompilerParams(
            dimension_semantics=("parallel", "parallel")),
    )(pmax, vals, rmask)
    return out, mask_out


# speedup vs baseline: 1.7688x; 1.0344x over previous
import numpy as np

import jax
import jax.numpy as jnp
from jax import lax
from jax.experimental import pallas as pl
from jax.experimental.pallas import tpu as pltpu

B, T, N = 16, 2048, 512
TC = 512
S = T // TC
NG = 6
GB = 3


def _s32(v):
    return int(np.int32(np.uint32(v)))


_K1 = (_s32(0x6D3E048F), _s32(0x1022172D))
_K2 = (_s32(0x03D7B32D), _s32(0xADD083F4))
_K3 = (_s32(0x92FB20EA), _s32(0x0F38D913))
_K4 = (_s32(0xBAD56946), _s32(0x354BA891))

_KP_MASK = 2516583
_KP_ZERO = 6710887
_KP_RND = 838861

_ROT_A = (13, 15, 26, 6)
_ROT_B = (17, 29, 16, 24)


def _rotl(x, r):
    return lax.shift_left(x, jnp.int32(r)) | lax.shift_right_logical(
        x, jnp.int32(32 - r))


def _wrap32(v):
    return int(np.int32(np.uint32(v & 0xFFFFFFFF)))


def _threefry(kpair, ctr):
    k0 = int(np.uint32(np.int32(kpair[0])))
    k1 = int(np.uint32(np.int32(kpair[1])))
    ks = (k0, k1, k0 ^ k1 ^ 0x1BD11BDA)
    x0 = jnp.full_like(ctr, jnp.int32(_wrap32(ks[0])))
    x1 = ctr + jnp.int32(_wrap32(ks[1]))
    rots = (_ROT_A, _ROT_B, _ROT_A, _ROT_B, _ROT_A)
    for i in range(5):
        for r in rots[i]:
            x0 = x0 + x1
            x1 = _rotl(x1, r)
            x1 = x1 ^ x0
        x0 = x0 + jnp.int32(_wrap32(ks[(i + 1) % 3]))
        x1 = x1 + jnp.int32(_wrap32(ks[(i + 2) % 3] + i + 1))
    return x0 ^ x1


def _bern(bits, k):
    return lax.shift_right_logical(bits, jnp.int32(9)) < jnp.int32(k)


def _u01(bits):
    f = lax.bitcast_convert_type(
        lax.shift_right_logical(bits, jnp.int32(9)) | jnp.int32(0x3F800000),
        jnp.float32)
    return f - jnp.float32(1.0)


SUB = 8
GRP = 32


def _t1_maskrow(mr_ref):
    c = (lax.broadcasted_iota(jnp.int32, (B, N), 0) * jnp.int32(N)
         + lax.broadcasted_iota(jnp.int32, (B, N), 1))
    mr_ref[...] = _bern(_threefry(_K1, c), _KP_MASK).astype(jnp.int32)


def _t2_tables(mr_ref, cbn_ref, pos_ref):
    for g in range(NG):
        base = jnp.int32(g * GB * T * N)

        def fill(p, _):
            cbn_ref[g, p] = base
            return 0

        lax.fori_loop(0, N, fill, 0)
        cnt = jnp.int32(0)
        for b in range(g * GB, min((g + 1) * GB, B)):
            cb = b * T * N

            def body(n, cnt):
                slot = jnp.minimum(cnt, jnp.int32(N - 1))
                cbn_ref[g, slot] = jnp.int32(cb) + n
                pos_ref[b, n] = slot
                return cnt + mr_ref[b, n]

            cnt = lax.fori_loop(0, N, body, cnt)


def _pass_a(cbn_ref, zrc_ref, u_ref):
    s = pl.program_id(1)
    cbn = jnp.broadcast_to(cbn_ref[0, :, :], (SUB, N))
    rowoff = lax.broadcasted_iota(jnp.int32, (SUB, N), 0) * jnp.int32(N)

    def slice_step(j, _):
        t0 = s * TC + j * SUB
        c = cbn + (t0 * jnp.int32(N)) + rowoff
        zero = _bern(_threefry(_K2, c), _KP_ZERO)
        rnd = _bern(_threefry(_K3, c), _KP_RND)
        u = _u01(_threefry(_K4, c))
        sl = pl.ds(j * SUB, SUB)
        zrc_ref[0, sl, :] = zero.astype(jnp.int32) + 2 * rnd.astype(jnp.int32)
        u_ref[0, sl, :] = u
        return 0

    lax.fori_loop(0, TC // SUB, slice_step, 0)


def _gather512(src, lanes, his):
    srcs = [src[:, 128 * k:128 * (k + 1)] for k in range(4)]
    outs = []
    for i in range(4):
        parts = [jnp.take_along_axis(sk, lanes[i], axis=1) for sk in srcs]
        hi = his[i]
        r = jnp.where(hi == 0, parts[0],
                      jnp.where(hi == 1, parts[1],
                                jnp.where(hi == 2, parts[2], parts[3])))
        outs.append(r)
    return jnp.concatenate(outs, axis=1)


def _pass_b(x_ref, mr_ref, pos_ref, zrc_ref, u_ref,
            vals_ref, maskout_ref, rmask_ref, pmax_ref):
    mask_sub = jnp.broadcast_to(mr_ref[0, :, :] != 0, (SUB, N))
    mask_i32 = mask_sub.astype(jnp.int32)
    pos = pos_ref[0, :, :]
    lanes, his = [], []
    for i in range(4):
        p = jnp.broadcast_to(pos[:, 128 * i:128 * (i + 1)], (SUB, 128))
        lanes.append(p & jnp.int32(127))
        his.append(lax.shift_right_logical(p, jnp.int32(7)))

    def group(j, mx):
        rnd_pieces = []
        for k in range(GRP // SUB):
            sl = pl.ds(j * GRP + k * SUB, SUB)
            x = x_ref[0, sl, :]
            code = _gather512(zrc_ref[0, sl, :], lanes, his)
            ug = _gather512(u_ref[0, sl, :], lanes, his)
            zero = ((code & 1) != 0) & mask_sub
            rnd = (code >= 2) & mask_sub & (~zero)
            feats = jnp.where(zero, jnp.float32(0.0), x)
            vals_ref[0, sl, :] = jnp.where(rnd, ug, feats)
            maskout_ref[0, sl, :] = mask_i32
            rnd_pieces.append(rnd)
            mx = jnp.maximum(mx, feats)
        rnd_grp = jnp.concatenate(rnd_pieces, axis=0)
        rmask_ref[0, pl.ds(j * GRP, GRP), :] = rnd_grp.astype(jnp.int8)
        return mx

    mx = lax.fori_loop(0, TC // GRP, group,
                       jnp.full((SUB, N), -jnp.inf, jnp.float32))
    pmax_ref[...] = jnp.full((1, 1, 1), jnp.max(mx), jnp.float32)


def _pass_c(pmax_ref, vals_ref, rmask_ref, out_ref):
    m = jnp.max(pmax_ref[...])
    v = vals_ref[...]
    spike = jnp.round(m * v)
    out_ref[...] = jnp.where(rmask_ref[...] != 0, spike, v)


def kernel(features):
    mr = pl.pallas_call(
        _t1_maskrow,
        out_specs=pl.BlockSpec((B, N), lambda: (0, 0)),
        out_shape=jax.ShapeDtypeStruct((B, N), jnp.int32),
    )()
    cbn, pos = pl.pallas_call(
        _t2_tables,
        in_specs=[pl.BlockSpec(memory_space=pltpu.SMEM)],
        out_specs=[
            pl.BlockSpec(memory_space=pltpu.SMEM),
            pl.BlockSpec(memory_space=pltpu.SMEM),
        ],
        out_shape=[
            jax.ShapeDtypeStruct((NG, N), jnp.int32),
            jax.ShapeDtypeStruct((B, N), jnp.int32),
        ],
    )(mr)
    cbn3 = cbn.reshape(NG, 1, N)
    pos3 = pos.reshape(B, 1, N)
    mr3 = mr.reshape(B, 1, N)
    zrc, u_c = pl.pallas_call(
        _pass_a,
        grid=(NG, S),
        in_specs=[pl.BlockSpec((1, 1, N), lambda g, s: (g, 0, 0))],
        out_specs=[
            pl.BlockSpec((1, TC, N), lambda g, s: (g, s, 0)),
            pl.BlockSpec((1, TC, N), lambda g, s: (g, s, 0)),
        ],
        out_shape=[
            jax.ShapeDtypeStruct((NG, T, N), jnp.int32),
            jax.ShapeDtypeStruct((NG, T, N), jnp.float32),
        ],
        compiler_params=pltpu.CompilerParams(
            dimension_semantics=("parallel", "parallel")),
    )(cbn3)
    blk = lambda b, s: (b, s, 0)
    vals, mask_out, rmask, pmax = pl.pallas_call(
        _pass_b,
        grid=(B, S),
        in_specs=[
            pl.BlockSpec((1, TC, N), blk),
            pl.BlockSpec((1, 1, N), lambda b, s: (b, 0, 0)),
            pl.BlockSpec((1, 1, N), lambda b, s: (b, 0, 0)),
            pl.BlockSpec((1, TC, N), lambda b, s: (b // GB, s, 0)),
            pl.BlockSpec((1, TC, N), lambda b, s: (b // GB, s, 0)),
        ],
        out_specs=[
            pl.BlockSpec((1, TC, N), blk),
            pl.BlockSpec((1, TC, N), blk),
            pl.BlockSpec((1, TC, N), blk),
            pl.BlockSpec((1, 1, 1), lambda b, s: (b * S + s, 0, 0)),
        ],
        out_shape=[
            jax.ShapeDtypeStruct((B, T, N), jnp.float32),
            jax.ShapeDtypeStruct((B, T, N), jnp.int32),
            jax.ShapeDtypeStruct((B, T, N), jnp.int8),
            jax.ShapeDtypeStruct((B * S, 1, 1), jnp.float32),
        ],
        compiler_params=pltpu.CompilerParams(
            dimension_semantics=("parallel", "parallel")),
    )(features, mr3, pos3, zrc, u_c)
    out = pl.pallas_call(
        _pass_c,
        grid=(B, S),
        in_specs=[
            pl.BlockSpec((B * S, 1, 1), lambda b, s: (0, 0, 0)),
            pl.BlockSpec((1, TC, N), blk),
            pl.BlockSpec((1, TC, N), blk),
        ],
        out_specs=pl.BlockSpec((1, TC, N), blk),
        out_shape=jax.ShapeDtypeStruct((B, T, N), jnp.float32),
        input_output_aliases={1: 0},
        compiler_params=pltpu.CompilerParams(
            dimension_semantics=("parallel", "parallel")),
    )(pmax, vals, rmask)
    return out, mask_out


# speedup vs baseline: 2.0651x; 1.1675x over previous
import numpy as np

import jax
import jax.numpy as jnp
from jax import lax
from jax.experimental import pallas as pl
from jax.experimental.pallas import tpu as pltpu

B, T, N = 16, 2048, 512
TC = 512
S = T // TC
NG = 6
GB = 3


def _s32(v):
    return int(np.int32(np.uint32(v)))


_K1 = (_s32(0x6D3E048F), _s32(0x1022172D))
_K2 = (_s32(0x03D7B32D), _s32(0xADD083F4))
_K3 = (_s32(0x92FB20EA), _s32(0x0F38D913))
_K4 = (_s32(0xBAD56946), _s32(0x354BA891))

_KP_MASK = 2516583
_KP_ZERO = 6710887
_KP_RND = 838861

_ROT_A = (13, 15, 26, 6)
_ROT_B = (17, 29, 16, 24)


def _rotl(x, r):
    return lax.shift_left(x, jnp.int32(r)) | lax.shift_right_logical(
        x, jnp.int32(32 - r))


def _wrap32(v):
    return int(np.int32(np.uint32(v & 0xFFFFFFFF)))


def _threefry(kpair, ctr):
    k0 = int(np.uint32(np.int32(kpair[0])))
    k1 = int(np.uint32(np.int32(kpair[1])))
    ks = (k0, k1, k0 ^ k1 ^ 0x1BD11BDA)
    x0 = jnp.full_like(ctr, jnp.int32(_wrap32(ks[0])))
    x1 = ctr + jnp.int32(_wrap32(ks[1]))
    rots = (_ROT_A, _ROT_B, _ROT_A, _ROT_B, _ROT_A)
    for i in range(5):
        for r in rots[i]:
            x0 = x0 + x1
            x1 = _rotl(x1, r)
            x1 = x1 ^ x0
        x0 = x0 + jnp.int32(_wrap32(ks[(i + 1) % 3]))
        x1 = x1 + jnp.int32(_wrap32(ks[(i + 2) % 3] + i + 1))
    return x0 ^ x1


def _bern(bits, k):
    return lax.shift_right_logical(bits, jnp.int32(9)) < jnp.int32(k)


def _u01(bits):
    f = lax.bitcast_convert_type(
        lax.shift_right_logical(bits, jnp.int32(9)) | jnp.int32(0x3F800000),
        jnp.float32)
    return f - jnp.float32(1.0)


SUB = 8
GRP = 32


def _t1_maskrow(mr_ref):
    c = (lax.broadcasted_iota(jnp.int32, (B, N), 0) * jnp.int32(N)
         + lax.broadcasted_iota(jnp.int32, (B, N), 1))
    mr_ref[...] = _bern(_threefry(_K1, c), _KP_MASK).astype(jnp.int32)


def _t2_tables(mr_ref, cbn_ref, pos_ref):
    for g in range(NG):
        base = jnp.int32(g * GB * T * N)

        def fill(p, _):
            cbn_ref[g, p] = base
            return 0

        lax.fori_loop(0, N, fill, 0)
        cnt = jnp.int32(0)
        for b in range(g * GB, min((g + 1) * GB, B)):
            cb = b * T * N

            def body(n, cnt):
                slot = jnp.minimum(cnt, jnp.int32(N - 1))
                cbn_ref[g, slot] = jnp.int32(cb) + n
                pos_ref[b, n] = slot
                return cnt + mr_ref[b, n]

            cnt = lax.fori_loop(0, N, body, cnt)


def _pass_a(cbn_ref, pk_ref):
    s = pl.program_id(1)
    cbn = jnp.broadcast_to(cbn_ref[0, :, :], (SUB, N))
    rowoff = lax.broadcasted_iota(jnp.int32, (SUB, N), 0) * jnp.int32(N)

    def slice_step(j, _):
        t0 = s * TC + j * SUB
        c = cbn + (t0 * jnp.int32(N)) + rowoff
        zero = _bern(_threefry(_K2, c), _KP_ZERO)
        rnd = _bern(_threefry(_K3, c), _KP_RND)
        ub9 = lax.shift_right_logical(_threefry(_K4, c), jnp.int32(9))
        pk = (lax.shift_left(ub9, jnp.int32(2))
              | jnp.where(zero, jnp.int32(1), jnp.int32(0))
              | jnp.where(rnd, jnp.int32(2), jnp.int32(0)))
        pk_ref[0, pl.ds(j * SUB, SUB), :] = pk
        return 0

    lax.fori_loop(0, TC // SUB, slice_step, 0)


def _gather512(src, lanes, his):
    srcs = [src[:, 128 * k:128 * (k + 1)] for k in range(4)]
    outs = []
    for i in range(4):
        parts = [jnp.take_along_axis(sk, lanes[i], axis=1) for sk in srcs]
        hi = his[i]
        r = jnp.where(hi == 0, parts[0],
                      jnp.where(hi == 1, parts[1],
                                jnp.where(hi == 2, parts[2], parts[3])))
        outs.append(r)
    return jnp.concatenate(outs, axis=1)


def _pass_b(x_ref, mr_ref, pos_ref, pk_ref,
            vals_ref, maskout_ref, rmask_ref, pmax_ref):
    mask_sub = jnp.broadcast_to(mr_ref[0, :, :] != 0, (SUB, N))
    mask_i32 = mask_sub.astype(jnp.int32)
    pos = pos_ref[0, :, :]
    lanes, his = [], []
    for i in range(4):
        p = jnp.broadcast_to(pos[:, 128 * i:128 * (i + 1)], (SUB, 128))
        lanes.append(p & jnp.int32(127))
        his.append(lax.shift_right_logical(p, jnp.int32(7)))

    def group(j, mx):
        rnd_pieces = []
        for k in range(GRP // SUB):
            sl = pl.ds(j * GRP + k * SUB, SUB)
            x = x_ref[0, sl, :]
            pk = _gather512(pk_ref[0, sl, :], lanes, his)
            ug = lax.bitcast_convert_type(
                lax.shift_right_logical(pk, jnp.int32(2))
                | jnp.int32(0x3F800000), jnp.float32) - jnp.float32(1.0)
            zero = ((pk & 1) != 0) & mask_sub
            rnd = ((pk & 2) != 0) & mask_sub & (~zero)
            feats = jnp.where(zero, jnp.float32(0.0), x)
            vals_ref[0, sl, :] = jnp.where(rnd, ug, feats)
            maskout_ref[0, sl, :] = mask_i32
            rnd_pieces.append(rnd)
            mx = jnp.maximum(mx, feats)
        rnd_grp = jnp.concatenate(rnd_pieces, axis=0)
        rmask_ref[0, pl.ds(j * GRP, GRP), :] = rnd_grp.astype(jnp.int8)
        return mx

    mx = lax.fori_loop(0, TC // GRP, group,
                       jnp.full((SUB, N), -jnp.inf, jnp.float32))
    pmax_ref[...] = jnp.full((1, 1, 1), jnp.max(mx), jnp.float32)


def _pass_c(pmax_ref, vals_ref, rmask_ref, out_ref):
    m = jnp.max(pmax_ref[...])
    v = vals_ref[...]
    spike = jnp.round(m * v)
    out_ref[...] = jnp.where(rmask_ref[...] != 0, spike, v)


def kernel(features):
    mr = pl.pallas_call(
        _t1_maskrow,
        out_specs=pl.BlockSpec((B, N), lambda: (0, 0)),
        out_shape=jax.ShapeDtypeStruct((B, N), jnp.int32),
    )()
    cbn, pos = pl.pallas_call(
        _t2_tables,
        in_specs=[pl.BlockSpec(memory_space=pltpu.SMEM)],
        out_specs=[
            pl.BlockSpec(memory_space=pltpu.SMEM),
            pl.BlockSpec(memory_space=pltpu.SMEM),
        ],
        out_shape=[
            jax.ShapeDtypeStruct((NG, N), jnp.int32),
            jax.ShapeDtypeStruct((B, N), jnp.int32),
        ],
    )(mr)
    cbn3 = cbn.reshape(NG, 1, N)
    pos3 = pos.reshape(B, 1, N)
    mr3 = mr.reshape(B, 1, N)
    pk = pl.pallas_call(
        _pass_a,
        grid=(NG, S),
        in_specs=[pl.BlockSpec((1, 1, N), lambda g, s: (g, 0, 0))],
        out_specs=pl.BlockSpec((1, TC, N), lambda g, s: (g, s, 0)),
        out_shape=jax.ShapeDtypeStruct((NG, T, N), jnp.int32),
        compiler_params=pltpu.CompilerParams(
            dimension_semantics=("parallel", "parallel")),
    )(cbn3)
    blk = lambda b, s: (b, s, 0)
    vals, mask_out, rmask, pmax = pl.pallas_call(
        _pass_b,
        grid=(B, S),
        in_specs=[
            pl.BlockSpec((1, TC, N), blk),
            pl.BlockSpec((1, 1, N), lambda b, s: (b, 0, 0)),
            pl.BlockSpec((1, 1, N), lambda b, s: (b, 0, 0)),
            pl.BlockSpec((1, TC, N), lambda b, s: (b // GB, s, 0)),
        ],
        out_specs=[
            pl.BlockSpec((1, TC, N), blk),
            pl.BlockSpec((1, TC, N), blk),
            pl.BlockSpec((1, TC, N), blk),
            pl.BlockSpec((1, 1, 1), lambda b, s: (b * S + s, 0, 0)),
        ],
        out_shape=[
            jax.ShapeDtypeStruct((B, T, N), jnp.float32),
            jax.ShapeDtypeStruct((B, T, N), jnp.int32),
            jax.ShapeDtypeStruct((B, T, N), jnp.int8),
            jax.ShapeDtypeStruct((B * S, 1, 1), jnp.float32),
        ],
        compiler_params=pltpu.CompilerParams(
            dimension_semantics=("parallel", "parallel")),
    )(features, mr3, pos3, pk)
    out = pl.pallas_call(
        _pass_c,
        grid=(B, S),
        in_specs=[
            pl.BlockSpec((B * S, 1, 1), lambda b, s: (0, 0, 0)),
            pl.BlockSpec((1, TC, N), blk),
            pl.BlockSpec((1, TC, N), blk),
        ],
        out_specs=pl.BlockSpec((1, TC, N), blk),
        out_shape=jax.ShapeDtypeStruct((B, T, N), jnp.float32),
        input_output_aliases={1: 0},
        compiler_params=pltpu.CompilerParams(
            dimension_semantics=("parallel", "parallel")),
    )(pmax, vals, rmask)
    return out, mask_out


# speedup vs baseline: 2.1126x; 1.0230x over previous
import numpy as np

import jax
import jax.numpy as jnp
from jax import lax
from jax.experimental import pallas as pl
from jax.experimental.pallas import tpu as pltpu

B, T, N = 16, 2048, 512
TC = 512
S = T // TC
NG = 6
GB = 3


def _s32(v):
    return int(np.int32(np.uint32(v)))


_K1 = (_s32(0x6D3E048F), _s32(0x1022172D))
_K2 = (_s32(0x03D7B32D), _s32(0xADD083F4))
_K3 = (_s32(0x92FB20EA), _s32(0x0F38D913))
_K4 = (_s32(0xBAD56946), _s32(0x354BA891))

_KP_MASK = 2516583
_KP_ZERO = 6710887
_KP_RND = 838861

_ROT_A = (13, 15, 26, 6)
_ROT_B = (17, 29, 16, 24)


def _rotl(x, r):
    return lax.shift_left(x, jnp.int32(r)) | lax.shift_right_logical(
        x, jnp.int32(32 - r))


def _wrap32(v):
    return int(np.int32(np.uint32(v & 0xFFFFFFFF)))


def _threefry(kpair, ctr):
    k0 = int(np.uint32(np.int32(kpair[0])))
    k1 = int(np.uint32(np.int32(kpair[1])))
    ks = (k0, k1, k0 ^ k1 ^ 0x1BD11BDA)
    x0 = jnp.full_like(ctr, jnp.int32(_wrap32(ks[0])))
    x1 = ctr + jnp.int32(_wrap32(ks[1]))
    rots = (_ROT_A, _ROT_B, _ROT_A, _ROT_B, _ROT_A)
    for i in range(5):
        for r in rots[i]:
            x0 = x0 + x1
            x1 = _rotl(x1, r)
            x1 = x1 ^ x0
        x0 = x0 + jnp.int32(_wrap32(ks[(i + 1) % 3]))
        x1 = x1 + jnp.int32(_wrap32(ks[(i + 2) % 3] + i + 1))
    return x0 ^ x1


def _bern(bits, k):
    return lax.shift_right_logical(bits, jnp.int32(9)) < jnp.int32(k)


def _u01(bits):
    f = lax.bitcast_convert_type(
        lax.shift_right_logical(bits, jnp.int32(9)) | jnp.int32(0x3F800000),
        jnp.float32)
    return f - jnp.float32(1.0)


SUB = 8
GRP = 32


def _t1_maskrow(mr_ref):
    c = (lax.broadcasted_iota(jnp.int32, (B, N), 0) * jnp.int32(N)
         + lax.broadcasted_iota(jnp.int32, (B, N), 1))
    mr_ref[...] = _bern(_threefry(_K1, c), _KP_MASK).astype(jnp.int32)


def _t2_tables(mr_ref, cbn_ref, pos_ref):
    for g in range(NG):
        base = jnp.int32(g * GB * T * N)

        def fill(p, _):
            cbn_ref[g, p] = base
            return 0

        lax.fori_loop(0, N, fill, 0)
        cnt = jnp.int32(0)
        for b in range(g * GB, min((g + 1) * GB, B)):
            cb = b * T * N

            def body(n, cnt):
                slot = jnp.minimum(cnt, jnp.int32(N - 1))
                cbn_ref[g, slot] = jnp.int32(cb) + n
                pos_ref[b, n] = slot
                return cnt + mr_ref[b, n]

            cnt = lax.fori_loop(0, N, body, cnt)


def _pass_a(cbn_ref, pk_ref):
    s = pl.program_id(1)
    cbn = jnp.broadcast_to(cbn_ref[0, :, :], (SUB, N))
    rowoff = lax.broadcasted_iota(jnp.int32, (SUB, N), 0) * jnp.int32(N)

    def slice_step(j, _):
        for k in range(2):
            t0 = s * TC + (j * 2 + k) * SUB
            c = cbn + (t0 * jnp.int32(N)) + rowoff
            zero = _bern(_threefry(_K2, c), _KP_ZERO)
            rnd = _bern(_threefry(_K3, c), _KP_RND)
            ub9 = lax.shift_right_logical(_threefry(_K4, c), jnp.int32(9))
            pk = (lax.shift_left(ub9, jnp.int32(2))
                  | jnp.where(zero, jnp.int32(1), jnp.int32(0))
                  | jnp.where(rnd, jnp.int32(2), jnp.int32(0)))
            pk_ref[0, pl.ds((j * 2 + k) * SUB, SUB), :] = pk
        return 0

    lax.fori_loop(0, TC // (2 * SUB), slice_step, 0)


def _gather512(src, lanes, his):
    srcs = [src[:, 128 * k:128 * (k + 1)] for k in range(4)]
    outs = []
    for i in range(4):
        parts = [jnp.take_along_axis(sk, lanes[i], axis=1) for sk in srcs]
        hi = his[i]
        r = jnp.where(hi == 0, parts[0],
                      jnp.where(hi == 1, parts[1],
                                jnp.where(hi == 2, parts[2], parts[3])))
        outs.append(r)
    return jnp.concatenate(outs, axis=1)


def _pass_b(x_ref, mr_ref, pos_ref, pk_ref,
            vals_ref, maskout_ref, rmask_ref, pmax_ref):
    mask_sub = jnp.broadcast_to(mr_ref[0, :, :] != 0, (SUB, N))
    mask_i32 = mask_sub.astype(jnp.int32)
    pos = pos_ref[0, :, :]
    lanes, his = [], []
    for i in range(4):
        p = jnp.broadcast_to(pos[:, 128 * i:128 * (i + 1)], (SUB, 128))
        lanes.append(p & jnp.int32(127))
        his.append(lax.shift_right_logical(p, jnp.int32(7)))

    def group(j, mx):
        rnd_pieces = []
        for k in range(GRP // SUB):
            sl = pl.ds(j * GRP + k * SUB, SUB)
            x = x_ref[0, sl, :]
            pk = _gather512(pk_ref[0, sl, :], lanes, his)
            ug = lax.bitcast_convert_type(
                lax.shift_right_logical(pk, jnp.int32(2))
                | jnp.int32(0x3F800000), jnp.float32) - jnp.float32(1.0)
            zero = ((pk & 1) != 0) & mask_sub
            rnd = ((pk & 2) != 0) & mask_sub & (~zero)
            feats = jnp.where(zero, jnp.float32(0.0), x)
            vals_ref[0, sl, :] = jnp.where(rnd, ug, feats)
            maskout_ref[0, sl, :] = mask_i32
            rnd_pieces.append(rnd)
            mx = jnp.maximum(mx, feats)
        rnd_grp = jnp.concatenate(rnd_pieces, axis=0)
        rmask_ref[0, pl.ds(j * GRP, GRP), :] = rnd_grp.astype(jnp.int8)
        return mx

    mx = lax.fori_loop(0, TC // GRP, group,
                       jnp.full((SUB, N), -jnp.inf, jnp.float32))
    pmax_ref[...] = jnp.full((1, 1, 1), jnp.max(mx), jnp.float32)


def _pass_c(pmax_ref, vals_ref, rmask_ref, out_ref):
    m = jnp.max(pmax_ref[...])
    v = vals_ref[...]
    spike = jnp.round(m * v)
    out_ref[...] = jnp.where(rmask_ref[...] != 0, spike, v)


def kernel(features):
    mr = pl.pallas_call(
        _t1_maskrow,
        out_specs=pl.BlockSpec((B, N), lambda: (0, 0)),
        out_shape=jax.ShapeDtypeStruct((B, N), jnp.int32),
    )()
    cbn, pos = pl.pallas_call(
        _t2_tables,
        in_specs=[pl.BlockSpec(memory_space=pltpu.SMEM)],
        out_specs=[
            pl.BlockSpec(memory_space=pltpu.SMEM),
            pl.BlockSpec(memory_space=pltpu.SMEM),
        ],
        out_shape=[
            jax.ShapeDtypeStruct((NG, N), jnp.int32),
            jax.ShapeDtypeStruct((B, N), jnp.int32),
        ],
    )(mr)
    cbn3 = cbn.reshape(NG, 1, N)
    pos3 = pos.reshape(B, 1, N)
    mr3 = mr.reshape(B, 1, N)
    pk = pl.pallas_call(
        _pass_a,
        grid=(NG, S),
        in_specs=[pl.BlockSpec((1, 1, N), lambda g, s: (g, 0, 0))],
        out_specs=pl.BlockSpec((1, TC, N), lambda g, s: (g, s, 0)),
        out_shape=jax.ShapeDtypeStruct((NG, T, N), jnp.int32),
        compiler_params=pltpu.CompilerParams(
            dimension_semantics=("parallel", "parallel")),
    )(cbn3)
    blk = lambda b, s: (b, s, 0)
    vals, mask_out, rmask, pmax = pl.pallas_call(
        _pass_b,
        grid=(B, S),
        in_specs=[
            pl.BlockSpec((1, TC, N), blk),
            pl.BlockSpec((1, 1, N), lambda b, s: (b, 0, 0)),
            pl.BlockSpec((1, 1, N), lambda b, s: (b, 0, 0)),
            pl.BlockSpec((1, TC, N), lambda b, s: (b // GB, s, 0)),
        ],
        out_specs=[
            pl.BlockSpec((1, TC, N), blk),
            pl.BlockSpec((1, TC, N), blk),
            pl.BlockSpec((1, TC, N), blk),
            pl.BlockSpec((1, 1, 1), lambda b, s: (b * S + s, 0, 0)),
        ],
        out_shape=[
            jax.ShapeDtypeStruct((B, T, N), jnp.float32),
            jax.ShapeDtypeStruct((B, T, N), jnp.int32),
            jax.ShapeDtypeStruct((B, T, N), jnp.int8),
            jax.ShapeDtypeStruct((B * S, 1, 1), jnp.float32),
        ],
        compiler_params=pltpu.CompilerParams(
            dimension_semantics=("parallel", "parallel")),
    )(features, mr3, pos3, pk)
    out = pl.pallas_call(
        _pass_c,
        grid=(B, S),
        in_specs=[
            pl.BlockSpec((B * S, 1, 1), lambda b, s: (0, 0, 0)),
            pl.BlockSpec((1, TC, N), blk),
            pl.BlockSpec((1, TC, N), blk),
        ],
        out_specs=pl.BlockSpec((1, TC, N), blk),
        out_shape=jax.ShapeDtypeStruct((B, T, N), jnp.float32),
        input_output_aliases={1: 0},
        compiler_params=pltpu.CompilerParams(
            dimension_semantics=("parallel", "parallel")),
    )(pmax, vals, rmask)
    return out, mask_out


# speedup vs baseline: 2.3141x; 1.0954x over previous
import numpy as np

import jax
import jax.numpy as jnp
from jax import lax
from jax.experimental import pallas as pl
from jax.experimental.pallas import tpu as pltpu

B, T, N = 16, 2048, 512
TC = 512
S = T // TC
NG = 6
GB = 3


def _s32(v):
    return int(np.int32(np.uint32(v)))


_K1 = (_s32(0x6D3E048F), _s32(0x1022172D))
_K2 = (_s32(0x03D7B32D), _s32(0xADD083F4))
_K3 = (_s32(0x92FB20EA), _s32(0x0F38D913))
_K4 = (_s32(0xBAD56946), _s32(0x354BA891))

_KP_MASK = 2516583
_KP_ZERO = 6710887
_KP_RND = 838861

_ROT_A = (13, 15, 26, 6)
_ROT_B = (17, 29, 16, 24)


def _rotl(x, r):
    return lax.shift_left(x, jnp.int32(r)) | lax.shift_right_logical(
        x, jnp.int32(32 - r))


def _wrap32(v):
    return int(np.int32(np.uint32(v & 0xFFFFFFFF)))


def _threefry(kpair, ctr):
    k0 = int(np.uint32(np.int32(kpair[0])))
    k1 = int(np.uint32(np.int32(kpair[1])))
    ks = (k0, k1, k0 ^ k1 ^ 0x1BD11BDA)
    x0 = jnp.full_like(ctr, jnp.int32(_wrap32(ks[0])))
    x1 = ctr + jnp.int32(_wrap32(ks[1]))
    rots = (_ROT_A, _ROT_B, _ROT_A, _ROT_B, _ROT_A)
    for i in range(5):
        for r in rots[i]:
            x0 = x0 + x1
            x1 = _rotl(x1, r)
            x1 = x1 ^ x0
        x0 = x0 + jnp.int32(_wrap32(ks[(i + 1) % 3]))
        x1 = x1 + jnp.int32(_wrap32(ks[(i + 2) % 3] + i + 1))
    return x0 ^ x1


def _bern(bits, k):
    return lax.shift_right_logical(bits, jnp.int32(9)) < jnp.int32(k)


def _u01(bits):
    f = lax.bitcast_convert_type(
        lax.shift_right_logical(bits, jnp.int32(9)) | jnp.int32(0x3F800000),
        jnp.float32)
    return f - jnp.float32(1.0)


SUB = 8
GRP = 64


def _t1_maskrow(mr_ref):
    c = (lax.broadcasted_iota(jnp.int32, (B, N), 0) * jnp.int32(N)
         + lax.broadcasted_iota(jnp.int32, (B, N), 1))
    mr_ref[...] = _bern(_threefry(_K1, c), _KP_MASK).astype(jnp.int32)


def _t2_tables(mr_ref, cbn_ref, pos_ref):
    for g in range(NG):
        cnt = jnp.int32(0)
        for b in range(g * GB, min((g + 1) * GB, B)):
            cb = b * T * N

            def body(n, cnt):
                slot = jnp.minimum(cnt, jnp.int32(N - 1))
                cbn_ref[g, slot] = jnp.int32(cb) + n
                pos_ref[b, n] = slot
                return cnt + mr_ref[b, n]

            cnt = lax.fori_loop(0, N, body, cnt)


def _pass_a(cbn_ref, pk_ref):
    s = pl.program_id(1)
    cbn = jnp.broadcast_to(cbn_ref[0, :, :], (SUB, N))
    rowoff = lax.broadcasted_iota(jnp.int32, (SUB, N), 0) * jnp.int32(N)

    def slice_step(j, _):
        for k in range(2):
            t0 = s * TC + (j * 2 + k) * SUB
            c = cbn + (t0 * jnp.int32(N)) + rowoff
            zero = _bern(_threefry(_K2, c), _KP_ZERO)
            rnd = _bern(_threefry(_K3, c), _KP_RND)
            ub9 = lax.shift_right_logical(_threefry(_K4, c), jnp.int32(9))
            pk = (lax.shift_left(ub9, jnp.int32(2))
                  | jnp.where(zero, jnp.int32(1), jnp.int32(0))
                  | jnp.where(rnd, jnp.int32(2), jnp.int32(0)))
            pk_ref[0, pl.ds((j * 2 + k) * SUB, SUB), :] = pk
        return 0

    lax.fori_loop(0, TC // (2 * SUB), slice_step, 0)


def _gather512(src, lanes, his):
    srcs = [src[:, 128 * k:128 * (k + 1)] for k in range(4)]
    outs = []
    for i in range(4):
        parts = [jnp.take_along_axis(sk, lanes[i], axis=1) for sk in srcs]
        hi = his[i]
        r = jnp.where(hi == 0, parts[0],
                      jnp.where(hi == 1, parts[1],
                                jnp.where(hi == 2, parts[2], parts[3])))
        outs.append(r)
    return jnp.concatenate(outs, axis=1)


def _pass_b(x_ref, mr_ref, pos_ref, pk_ref,
            vals_ref, maskout_ref, rmask_ref, pmax_ref):
    mask_sub = jnp.broadcast_to(mr_ref[0, :, :] != 0, (SUB, N))
    mask_i32 = mask_sub.astype(jnp.int32)
    pos = pos_ref[0, :, :]
    lanes, his = [], []
    for i in range(4):
        p = jnp.broadcast_to(pos[:, 128 * i:128 * (i + 1)], (SUB, 128))
        lanes.append(p & jnp.int32(127))
        his.append(lax.shift_right_logical(p, jnp.int32(7)))

    def group(j, mx):
        rnd_pieces = []
        for k in range(GRP // SUB):
            sl = pl.ds(j * GRP + k * SUB, SUB)
            x = x_ref[0, sl, :]
            pk = _gather512(pk_ref[0, sl, :], lanes, his)
            ug = lax.bitcast_convert_type(
                lax.shift_right_logical(pk, jnp.int32(2))
                | jnp.int32(0x3F800000), jnp.float32) - jnp.float32(1.0)
            zero = ((pk & 1) != 0) & mask_sub
            rnd = ((pk & 2) != 0) & mask_sub & (~zero)
            feats = jnp.where(zero, jnp.float32(0.0), x)
            vals_ref[0, sl, :] = jnp.where(rnd, ug, feats)
            maskout_ref[0, sl, :] = mask_i32
            rnd_pieces.append(rnd)
            mx = jnp.maximum(mx, feats)
        rnd_grp = jnp.concatenate(rnd_pieces, axis=0)
        rmask_ref[0, pl.ds(j * GRP, GRP), :] = rnd_grp.astype(jnp.int8)
        return mx

    mx = lax.fori_loop(0, TC // GRP, group,
                       jnp.full((SUB, N), -jnp.inf, jnp.float32))
    pmax_ref[...] = jnp.full((1, 1, 1), jnp.max(mx), jnp.float32)


def _pass_c(pmax_ref, vals_ref, rmask_ref, out_ref):
    m = jnp.max(pmax_ref[...])
    v = vals_ref[...]
    spike = jnp.round(m * v)
    out_ref[...] = jnp.where(rmask_ref[...] != 0, spike, v)


def kernel(features):
    mr = pl.pallas_call(
        _t1_maskrow,
        out_specs=pl.BlockSpec((B, N), lambda: (0, 0)),
        out_shape=jax.ShapeDtypeStruct((B, N), jnp.int32),
    )()
    cbn, pos = pl.pallas_call(
        _t2_tables,
        in_specs=[pl.BlockSpec(memory_space=pltpu.SMEM)],
        out_specs=[
            pl.BlockSpec(memory_space=pltpu.SMEM),
            pl.BlockSpec(memory_space=pltpu.SMEM),
        ],
        out_shape=[
            jax.ShapeDtypeStruct((NG, N), jnp.int32),
            jax.ShapeDtypeStruct((B, N), jnp.int32),
        ],
    )(mr)
    cbn3 = cbn.reshape(NG, 1, N)
    pos3 = pos.reshape(B, 1, N)
    mr3 = mr.reshape(B, 1, N)
    pk = pl.pallas_call(
        _pass_a,
        grid=(NG, S),
        in_specs=[pl.BlockSpec((1, 1, N), lambda g, s: (g, 0, 0))],
        out_specs=pl.BlockSpec((1, TC, N), lambda g, s: (g, s, 0)),
        out_shape=jax.ShapeDtypeStruct((NG, T, N), jnp.int32),
        compiler_params=pltpu.CompilerParams(
            dimension_semantics=("parallel", "parallel")),
    )(cbn3)
    blk = lambda b, s: (b, s, 0)
    vals, mask_out, rmask, pmax = pl.pallas_call(
        _pass_b,
        grid=(B, S),
        in_specs=[
            pl.BlockSpec((1, TC, N), blk),
            pl.BlockSpec((1, 1, N), lambda b, s: (b, 0, 0)),
            pl.BlockSpec((1, 1, N), lambda b, s: (b, 0, 0)),
            pl.BlockSpec((1, TC, N), lambda b, s: (b // GB, s, 0)),
        ],
        out_specs=[
            pl.BlockSpec((1, TC, N), blk),
            pl.BlockSpec((1, TC, N), blk),
            pl.BlockSpec((1, TC, N), blk),
            pl.BlockSpec((1, 1, 1), lambda b, s: (b * S + s, 0, 0)),
        ],
        out_shape=[
            jax.ShapeDtypeStruct((B, T, N), jnp.float32),
            jax.ShapeDtypeStruct((B, T, N), jnp.int32),
            jax.ShapeDtypeStruct((B, T, N), jnp.int8),
            jax.ShapeDtypeStruct((B * S, 1, 1), jnp.float32),
        ],
        compiler_params=pltpu.CompilerParams(
            dimension_semantics=("parallel", "parallel")),
    )(features, mr3, pos3, pk)
    out = pl.pallas_call(
        _pass_c,
        grid=(B, S),
        in_specs=[
            pl.BlockSpec((B * S, 1, 1), lambda b, s: (0, 0, 0)),
            pl.BlockSpec((1, TC, N), blk),
            pl.BlockSpec((1, TC, N), blk),
        ],
        out_specs=pl.BlockSpec((1, TC, N), blk),
        out_shape=jax.ShapeDtypeStruct((B, T, N), jnp.float32),
        input_output_aliases={1: 0},
        compiler_params=pltpu.CompilerParams(
            dimension_semantics=("parallel", "parallel")),
    )(pmax, vals, rmask)
    return out, mask_out
